# baseline (speedup 1.0000x reference)
import functools

import jax
import jax.numpy as jnp
from jax import lax
from jax.experimental import pallas as pl
from jax.experimental.pallas import tpu as pltpu
from jax.experimental.pallas import tpu_sc as plsc


def _combine_sc_body(ids_hbm, tw_hbm, out_hbm, i0_v, i1_v, w0_v, w1_v, out_v):
    T = 256
    wid = lax.axis_index("s") * 2 + lax.axis_index("c")

    @pl.when(wid < 16)
    def _():
        base = wid * 16
        pltpu.sync_copy(ids_hbm.at[pl.ds(base, 16)], i0_v)
        pltpu.sync_copy(ids_hbm.at[pl.ds(T + base, 16)], i1_v)
        pltpu.sync_copy(tw_hbm.at[pl.ds(base, 16)], w0_v)
        pltpu.sync_copy(tw_hbm.at[pl.ds(T + base, 16)], w1_v)
        id0, id1 = i0_v[...], i1_v[...]
        w0, w1 = w0_v[...], w1_v[...]
        fzero = jnp.zeros((16,), jnp.float32)
        for e in range(8):
            ev = jnp.full((16,), e, jnp.int32)
            out_v[pl.ds(e * 16, 16)] = (jnp.where(id0 == ev, w0, fzero)
                                        + jnp.where(id1 == ev, w1, fzero))
        for e in range(8):
            pltpu.sync_copy(out_v.at[pl.ds(e * 16, 16)],
                            out_hbm.at[pl.ds(e * T + base, 16)])


def _moe_kernel(x_ref, g_ref, u_ref, d_ref, c_ref, o_ref):
    e = pl.program_id(0)
    f = pl.program_id(1)

    @pl.when((e == 0) & (f == 0))
    def _init():
        o_ref[...] = jnp.zeros_like(o_ref)

    x = x_ref[...].astype(jnp.bfloat16)
    g = g_ref[0].astype(jnp.bfloat16)
    u = u_ref[0].astype(jnp.bfloat16)
    d = d_ref[0].astype(jnp.bfloat16)

    gate = jax.lax.dot_general(x, g, (((1,), (1,)), ((), ())),
                               preferred_element_type=jnp.float32)
    up = jax.lax.dot_general(x, u, (((1,), (1,)), ((), ())),
                             preferred_element_type=jnp.float32)
    act = (gate * jax.lax.logistic(gate)) * up
    eo = jax.lax.dot_general(act.astype(jnp.bfloat16), d,
                             (((1,), (1,)), ((), ())),
                             preferred_element_type=jnp.float32)

    cmb = c_ref[...]
    ecol = lax.broadcasted_iota(jnp.int32, cmb.shape, 1)
    w = jnp.sum(jnp.where(ecol == e, cmb, 0.0), axis=1)
    o_ref[...] += w[:, None] * eo


@functools.partial(jax.jit, static_argnames=())
def kernel(hidden_states, topk_weights, topk_ids, gate_up_weight, down_weight):
    T, D = hidden_states.shape
    E, two_ffn, _ = gate_up_weight.shape
    ffn = two_ffn // 2
    F = 1024
    nf = ffn // F

    mesh = plsc.VectorSubcoreMesh(core_axis_name="c", subcore_axis_name="s")
    combine = pl.kernel(
        _combine_sc_body,
        mesh=mesh,
        out_type=jax.ShapeDtypeStruct((E * T,), jnp.float32),
        scratch_types=[
            pltpu.VMEM((16,), jnp.int32),
            pltpu.VMEM((16,), jnp.int32),
            pltpu.VMEM((16,), jnp.float32),
            pltpu.VMEM((16,), jnp.float32),
            pltpu.VMEM((16 * E,), jnp.float32),
        ],
    )(topk_ids.T.reshape(-1), topk_weights.T.reshape(-1))
    combine = combine.reshape(E, T).T

    grid = (E, nf)
    out = pl.pallas_call(
        _moe_kernel,
        grid=grid,
        in_specs=[
            pl.BlockSpec((T, D), lambda e, f: (0, 0)),
            pl.BlockSpec((1, F, D), lambda e, f: (e, f, 0)),
            pl.BlockSpec((1, F, D), lambda e, f, _nf=nf: (e, f + _nf, 0)),
            pl.BlockSpec((1, D, F), lambda e, f: (e, 0, f)),
            pl.BlockSpec((T, E), lambda e, f: (0, 0)),
        ],
        out_specs=pl.BlockSpec((T, D), lambda e, f: (0, 0)),
        out_shape=jax.ShapeDtypeStruct((T, D), jnp.float32),
    )(hidden_states, gate_up_weight, gate_up_weight, down_weight, combine)
    return out


# speedup vs baseline: 1.3248x; 1.3248x over previous
import functools

import jax
import jax.numpy as jnp
from jax.experimental import pallas as pl


def _moe_kernel(x_ref, g_ref, u_ref, d_ref, tw_ref, ids_ref, o_ref):
    e = pl.program_id(0)
    f = pl.program_id(1)

    @pl.when((e == 0) & (f == 0))
    def _init():
        o_ref[...] = jnp.zeros_like(o_ref)

    x = x_ref[...].astype(jnp.bfloat16)
    g = g_ref[0].astype(jnp.bfloat16)
    u = u_ref[0].astype(jnp.bfloat16)
    d = d_ref[0].astype(jnp.bfloat16)

    gate = jax.lax.dot_general(x, g, (((1,), (1,)), ((), ())),
                               preferred_element_type=jnp.float32)
    up = jax.lax.dot_general(x, u, (((1,), (1,)), ((), ())),
                             preferred_element_type=jnp.float32)
    act = (gate * jax.lax.logistic(gate)) * up
    eo = jax.lax.dot_general(act.astype(jnp.bfloat16), d,
                             (((1,), (1,)), ((), ())),
                             preferred_element_type=jnp.float32)

    ids = ids_ref[...]
    tw = tw_ref[...]
    w = jnp.sum(jnp.where(ids == e, tw, 0.0), axis=1)
    o_ref[...] += w[:, None] * eo


@functools.partial(jax.jit, static_argnames=())
def kernel(hidden_states, topk_weights, topk_ids, gate_up_weight, down_weight):
    T, D = hidden_states.shape
    E, two_ffn, _ = gate_up_weight.shape
    ffn = two_ffn // 2
    F = 1024
    nf = ffn // F

    grid = (E, nf)
    out = pl.pallas_call(
        _moe_kernel,
        grid=grid,
        in_specs=[
            pl.BlockSpec((T, D), lambda e, f: (0, 0)),
            pl.BlockSpec((1, F, D), lambda e, f: (e, f, 0)),
            pl.BlockSpec((1, F, D), lambda e, f, _nf=nf: (e, f + _nf, 0)),
            pl.BlockSpec((1, D, F), lambda e, f: (e, 0, f)),
            pl.BlockSpec(topk_weights.shape, lambda e, f: (0, 0)),
            pl.BlockSpec(topk_ids.shape, lambda e, f: (0, 0)),
        ],
        out_specs=pl.BlockSpec((T, D), lambda e, f: (0, 0)),
        out_shape=jax.ShapeDtypeStruct((T, D), jnp.float32),
    )(hidden_states, gate_up_weight, gate_up_weight, down_weight,
      topk_weights, topk_ids)
    return out
